# baseline (speedup 1.0000x reference)
import functools

import jax
import jax.numpy as jnp
from jax import lax
from jax.experimental import pallas as pl
from jax.experimental.pallas import tpu as pltpu
from jax.experimental.pallas import tpu_sc as plsc

_B, _N, _D = 1024, 200, 128
_HIDDEN = 128
_INPUT_DIM = 256
_MAX_F = 200
_N_SESSIONS = 1000
_G = 64
_SCALE = float(_HIDDEN) ** 0.5

_NC, _NS, _L = 2, 16, 16
_NW = _NC * _NS
_BPW = _B // _NW
_HCH = _HIDDEN // _L


def _mlp_body(inp_ref, tokW_ref, tokb_ref, projW_ref, projb_ref, x_ref):
    inp = inp_ref[...].reshape(_G * _N, _D).astype(jnp.bfloat16)
    t = jnp.dot(inp, tokW_ref[...], preferred_element_type=jnp.float32)
    t = t + tokb_ref[...]
    t = (0.5 * t * (1.0 + lax.erf(t * (2.0 ** -0.5)))) * _SCALE
    y = jnp.dot(t.astype(jnp.bfloat16), projW_ref[...],
                preferred_element_type=jnp.float32)
    y = y + projb_ref[...]
    x_ref[...] = y.reshape(_G, _N, _HIDDEN)


def _mlp(inputs, tokWt, tokb2, projWt, projb2):
    return pl.pallas_call(
        _mlp_body,
        grid=(_B // _G,),
        in_specs=[
            pl.BlockSpec((_G, _N, _D), lambda i: (i, 0, 0)),
            pl.BlockSpec((_D, _INPUT_DIM), lambda i: (0, 0)),
            pl.BlockSpec((1, _INPUT_DIM), lambda i: (0, 0)),
            pl.BlockSpec((_INPUT_DIM, _HIDDEN), lambda i: (0, 0)),
            pl.BlockSpec((1, _HIDDEN), lambda i: (0, 0)),
        ],
        out_specs=pl.BlockSpec((_G, _N, _HIDDEN), lambda i: (i, 0, 0)),
        out_shape=jax.ShapeDtypeStruct((_B, _N, _HIDDEN), jnp.float32),
    )(inputs, tokWt, tokb2, projWt, projb2)


_NBUF = 2
_NPAD = 208


def _emb_body(posmod_hbm, ts_hbm, eid_hbm, sess_hbm, out_hbm,
              eid_v, sess_rows, table_v, ts_all, buf, sem_s, sem_w):
    wid = lax.axis_index("s") * _NC + lax.axis_index("c")
    base = wid * _BPW
    pltpu.sync_copy(posmod_hbm, table_v)
    pltpu.sync_copy(ts_hbm.at[pl.ds(base, _BPW)], ts_all)
    pltpu.sync_copy(eid_hbm.at[pl.ds(base, _BPW)], eid_v)
    pltpu.async_copy(sess_hbm.at[eid_v], sess_rows, sem_s).wait()

    def outer(jj, _):
        for s in range(_NBUF):
            j = jj * _NBUF + s
            b = base + j

            @pl.when(jj > 0)
            def _():
                pltpu.make_async_copy(buf.at[s], out_hbm.at[b - _NBUF],
                                      sem_w).wait()

            rows = [sess_rows[j, pl.ds(16 * k, 16)] for k in range(_HCH)]

            def do16(start):
                tsv = ts_all[j, pl.ds(start, 16)]
                for l in range(16):
                    tsn = tsv[l]
                    n = start + l
                    for k in range(_HCH):
                        sl = pl.ds(16 * k, 16)
                        buf[s, n, sl] = table_v[tsn, sl] + rows[k]

            def chunk(c, _):
                do16(pl.multiple_of(c * 16, 16))
                return 0

            lax.fori_loop(0, _N // 16, chunk, 0)
            do16(_N - 16)
            pltpu.async_copy(buf.at[s], out_hbm.at[b], sem_w)
        return 0

    lax.fori_loop(0, _BPW // _NBUF, outer, 0)
    for s in range(_NBUF):
        b_last = base + _BPW - _NBUF + s
        pltpu.make_async_copy(buf.at[s], out_hbm.at[b_last], sem_w).wait()


@functools.partial(
    pl.kernel,
    mesh=plsc.VectorSubcoreMesh(core_axis_name="c", subcore_axis_name="s"),
    out_type=jax.ShapeDtypeStruct((_B, _N, _HIDDEN), jnp.float32),
    scratch_types=[
        pltpu.VMEM((_BPW,), jnp.int32),
        pltpu.VMEM((_BPW, _HIDDEN), jnp.float32),
        pltpu.VMEM((_MAX_F, _HIDDEN), jnp.float32),
        pltpu.VMEM((_BPW, _N), jnp.int32),
        pltpu.VMEM((_NBUF, _N, _HIDDEN), jnp.float32),
        pltpu.SemaphoreType.DMA,
        pltpu.SemaphoreType.DMA,
    ],
)
def _emb_sc(posmod_hbm, ts_hbm, eid_hbm, sess_hbm, out_hbm,
            eid_v, sess_rows, table_v, ts_all, buf, sem_s, sem_w):
    _emb_body(posmod_hbm, ts_hbm, eid_hbm, sess_hbm, out_hbm,
              eid_v, sess_rows, table_v, ts_all, buf, sem_s, sem_w)


def kernel(inputs, inputs_timestamp, inputs_modality, eid, targets, tok_W,
           tok_b, proj_W, proj_b, mod_emb, pos_emb, sess_emb):
    tokWt = tok_W.T.astype(jnp.bfloat16)
    projWt = proj_W.T.astype(jnp.bfloat16)
    tokb2 = tok_b.reshape(1, _INPUT_DIM)
    projb2 = proj_b.reshape(1, _HIDDEN)
    posmod = pos_emb + mod_emb[inputs_modality][None, :]
    eid32 = eid.astype(jnp.int32)
    ts32 = inputs_timestamp.astype(jnp.int32)

    x = _mlp(inputs, tokWt, tokb2, projWt, projb2)
    emb = _emb_sc(posmod, ts32, eid32, sess_emb)
    return (x, emb, targets)


# speedup vs baseline: 1.4772x; 1.4772x over previous
import functools

import jax
import jax.numpy as jnp
from jax import lax
from jax.experimental import pallas as pl
from jax.experimental.pallas import tpu as pltpu
from jax.experimental.pallas import tpu_sc as plsc

_B, _N, _D = 1024, 200, 128
_HIDDEN = 128
_INPUT_DIM = 256
_MAX_F = 200
_N_SESSIONS = 1000
_G = 64
_SCALE = float(_HIDDEN) ** 0.5

_NC, _NS, _L = 2, 16, 16
_NW = _NC * _NS
_BPW = _B // _NW
_HCH = _HIDDEN // _L


def _mlp_body(inp_ref, tokW_ref, tokb_ref, projW_ref, projb_ref, x_ref):
    inp = inp_ref[...].reshape(_G * _N, _D).astype(jnp.bfloat16)
    t = jnp.dot(inp, tokW_ref[...], preferred_element_type=jnp.float32)
    t = t + tokb_ref[...]
    t = (0.5 * t * (1.0 + lax.erf(t * (2.0 ** -0.5)))) * _SCALE
    y = jnp.dot(t.astype(jnp.bfloat16), projW_ref[...],
                preferred_element_type=jnp.float32)
    y = y + projb_ref[...]
    x_ref[...] = y.reshape(_G, _N, _HIDDEN)


def _mlp(inputs, tokWt, tokb2, projWt, projb2):
    return pl.pallas_call(
        _mlp_body,
        grid=(_B // _G,),
        in_specs=[
            pl.BlockSpec((_G, _N, _D), lambda i: (i, 0, 0)),
            pl.BlockSpec((_D, _INPUT_DIM), lambda i: (0, 0)),
            pl.BlockSpec((1, _INPUT_DIM), lambda i: (0, 0)),
            pl.BlockSpec((_INPUT_DIM, _HIDDEN), lambda i: (0, 0)),
            pl.BlockSpec((1, _HIDDEN), lambda i: (0, 0)),
        ],
        out_specs=pl.BlockSpec((_G, _N, _HIDDEN), lambda i: (i, 0, 0)),
        out_shape=jax.ShapeDtypeStruct((_B, _N, _HIDDEN), jnp.float32),
    )(inputs, tokWt, tokb2, projWt, projb2)


_NBUF = 2
_NPAD = 208


def _emb_body(posmod_hbm, ts_hbm, eid_hbm, sess_hbm, out_hbm,
              eid_v, sess_rows, table_v, ts_all, buf, sem_s, sem_w):
    wid = lax.axis_index("s") * _NC + lax.axis_index("c")
    base = wid * _BPW
    pltpu.sync_copy(posmod_hbm, table_v)
    pltpu.sync_copy(ts_hbm.at[pl.ds(base, _BPW)], ts_all)
    pltpu.sync_copy(eid_hbm.at[pl.ds(base, _BPW)], eid_v)
    pltpu.async_copy(sess_hbm.at[eid_v], sess_rows, sem_s).wait()

    def outer(jj, _):
        for s in range(_NBUF):
            j = jj * _NBUF + s
            b = base + j

            @pl.when(jj > 0)
            def _():
                pltpu.make_async_copy(buf.at[s], out_hbm.at[b - _NBUF],
                                      sem_w).wait()

            rows = [sess_rows[j, pl.ds(16 * k, 16)] for k in range(_HCH)]

            def do16(start):
                tsv = ts_all[j, pl.ds(start, 16)]
                for l in range(16):
                    tsn = tsv[l]
                    n = start + l
                    for k in range(_HCH):
                        sl = pl.ds(16 * k, 16)
                        buf[s, n, sl] = table_v[tsn, sl] + rows[k]

            @plsc.parallel_loop(0, _N // 16, unroll=2)
            def _chunks(c):
                do16(pl.multiple_of(c * 16, 16))
            do16(_N - 16)
            pltpu.async_copy(buf.at[s], out_hbm.at[b], sem_w)
        return 0

    lax.fori_loop(0, _BPW // _NBUF, outer, 0)
    for s in range(_NBUF):
        b_last = base + _BPW - _NBUF + s
        pltpu.make_async_copy(buf.at[s], out_hbm.at[b_last], sem_w).wait()


@functools.partial(
    pl.kernel,
    mesh=plsc.VectorSubcoreMesh(core_axis_name="c", subcore_axis_name="s"),
    out_type=jax.ShapeDtypeStruct((_B, _N, _HIDDEN), jnp.float32),
    scratch_types=[
        pltpu.VMEM((_BPW,), jnp.int32),
        pltpu.VMEM((_BPW, _HIDDEN), jnp.float32),
        pltpu.VMEM((_MAX_F, _HIDDEN), jnp.float32),
        pltpu.VMEM((_BPW, _N), jnp.int32),
        pltpu.VMEM((_NBUF, _N, _HIDDEN), jnp.float32),
        pltpu.SemaphoreType.DMA,
        pltpu.SemaphoreType.DMA,
    ],
)
def _emb_sc(posmod_hbm, ts_hbm, eid_hbm, sess_hbm, out_hbm,
            eid_v, sess_rows, table_v, ts_all, buf, sem_s, sem_w):
    _emb_body(posmod_hbm, ts_hbm, eid_hbm, sess_hbm, out_hbm,
              eid_v, sess_rows, table_v, ts_all, buf, sem_s, sem_w)


def kernel(inputs, inputs_timestamp, inputs_modality, eid, targets, tok_W,
           tok_b, proj_W, proj_b, mod_emb, pos_emb, sess_emb):
    tokWt = tok_W.T.astype(jnp.bfloat16)
    projWt = proj_W.T.astype(jnp.bfloat16)
    tokb2 = tok_b.reshape(1, _INPUT_DIM)
    projb2 = proj_b.reshape(1, _HIDDEN)
    posmod = pos_emb + mod_emb[inputs_modality][None, :]
    eid32 = eid.astype(jnp.int32)
    ts32 = inputs_timestamp.astype(jnp.int32)

    x = _mlp(inputs, tokWt, tokb2, projWt, projb2)
    emb = _emb_sc(posmod, ts32, eid32, sess_emb)
    return (x, emb, targets)
